# Initial kernel scaffold; baseline (speedup 1.0000x reference)
import jax
import jax.numpy as jnp
from jax.experimental import pallas as pl


def kernel(Q, K, V):
    raise NotImplementedError("write your pallas kernel here")


# speedup vs baseline: 9.4955x; 9.4955x over previous
import math

import numpy as np
import jax
import jax.numpy as jnp
from jax import lax
from jax.experimental import pallas as pl
from jax.experimental.pallas import tpu as pltpu
from jax.experimental.pallas import tpu_sc as plsc

_L = 4096
_N = 64
_H = 12
_DM = 768
_DH2 = 128
_G = 4 * _H // 2
_K = 16
_NW = 32
_NCOLS = 4 * _DM
_CPW = _NCOLS // _NW

_tt = np.arange(_N)
_F = np.exp(-2j * np.pi * np.outer(_tt, _tt) / _N)
_T = np.exp(-2j * np.pi * np.outer(_tt, _tt) / _L)
_F1RE = np.ascontiguousarray(_F.real, np.float32)
_F1IM = np.ascontiguousarray(_F.imag, np.float32)
_TRE = np.ascontiguousarray(_T.real, np.float32)
_TIM = np.ascontiguousarray(_T.imag, np.float32)
_TTRE = np.ascontiguousarray(_T.real.T, np.float32)
_TTIM = np.ascontiguousarray(-_T.imag.T, np.float32)


def _fwd_fft(x, fre, fim, tre, tim):
    X = x.reshape(_N, _N * _DH2)
    yre = jnp.dot(fre, X, preferred_element_type=jnp.float32, precision=lax.Precision.HIGHEST)
    yim = jnp.dot(fim, X, preferred_element_type=jnp.float32, precision=lax.Precision.HIGHEST)
    y3re = yre.reshape(_N, _N, _DH2)
    y3im = yim.reshape(_N, _N, _DH2)
    t_re = tre[:, :, None]
    t_im = tim[:, :, None]
    zre = y3re * t_re - y3im * t_im
    zim = y3re * t_im + y3im * t_re
    zre = zre.transpose(1, 0, 2).reshape(_N, _N * _DH2)
    zim = zim.transpose(1, 0, 2).reshape(_N, _N * _DH2)
    wre = (jnp.dot(fre, zre, preferred_element_type=jnp.float32, precision=lax.Precision.HIGHEST)
           - jnp.dot(fim, zim, preferred_element_type=jnp.float32, precision=lax.Precision.HIGHEST))
    wim = (jnp.dot(fre, zim, preferred_element_type=jnp.float32, precision=lax.Precision.HIGHEST)
           + jnp.dot(fim, zre, preferred_element_type=jnp.float32, precision=lax.Precision.HIGHEST))
    return wre, wim


def _inv_fft_real(pre, pim, fre, fim, ttre, ttim):
    gre = (jnp.dot(fre, pre, preferred_element_type=jnp.float32, precision=lax.Precision.HIGHEST)
           + jnp.dot(fim, pim, preferred_element_type=jnp.float32, precision=lax.Precision.HIGHEST))
    gim = (jnp.dot(fre, pim, preferred_element_type=jnp.float32, precision=lax.Precision.HIGHEST)
           - jnp.dot(fim, pre, preferred_element_type=jnp.float32, precision=lax.Precision.HIGHEST))
    g3re = gre.reshape(_N, _N, _DH2)
    g3im = gim.reshape(_N, _N, _DH2)
    t_re = ttre[:, :, None]
    t_im = ttim[:, :, None]
    hre = g3re * t_re - g3im * t_im
    him = g3re * t_im + g3im * t_re
    hre = hre.transpose(1, 0, 2).reshape(_N, _N * _DH2)
    him = him.transpose(1, 0, 2).reshape(_N, _N * _DH2)
    rre = (jnp.dot(fre, hre, preferred_element_type=jnp.float32, precision=lax.Precision.HIGHEST)
           + jnp.dot(fim, him, preferred_element_type=jnp.float32, precision=lax.Precision.HIGHEST))
    return rre.reshape(_L, _DH2) * (1.0 / _L)


def _corr_topk_body(q_ref, k_ref, fre_ref, fim_ref, tre_ref, tim_ref,
                    ttre_ref, ttim_ref, w_ref, i_ref):
    fre = fre_ref[...]
    fim = fim_ref[...]
    tre = tre_ref[...]
    tim = tim_ref[...]
    ttre = ttre_ref[...]
    ttim = ttim_ref[...]

    q = q_ref[0]
    k = k_ref[0]
    qre, qim = _fwd_fft(q, fre, fim, tre, tim)
    kre, kim = _fwd_fft(k, fre, fim, tre, tim)
    pre = qre * kre + qim * kim
    pim = qim * kre - qre * kim
    corr = _inv_fft_real(pre, pim, fre, fim, ttre, ttim)

    iota_t = lax.broadcasted_iota(jnp.int32, (_L, _DH2), 0)
    c = corr
    wrows = []
    irows = []
    for _ in range(_K):
        m = jnp.max(c, axis=0, keepdims=True)
        hit = c == m
        idx = jnp.min(jnp.where(hit, iota_t, _L), axis=0, keepdims=True)
        wrows.append(m)
        irows.append(idx)
        c = jnp.where(iota_t == idx, -jnp.inf, c)
    wmat = jnp.concatenate(wrows, axis=0)
    imat = jnp.concatenate(irows, axis=0)
    wmax = jnp.max(wmat, axis=0, keepdims=True)
    e = jnp.exp(wmat - wmax)
    wsm = e / jnp.sum(e, axis=0, keepdims=True)
    w_ref[0] = wsm
    i_ref[0] = imat


def _make_corr_topk(interpret=False):
    hp = _H // 2
    const_spec = pl.BlockSpec((_N, _N), lambda g: (0, 0))
    return pl.pallas_call(
        _corr_topk_body,
        grid=(_G,),
        in_specs=[
            pl.BlockSpec((1, _L, _DH2), lambda g: (g // hp, 0, g % hp)),
            pl.BlockSpec((1, _L, _DH2), lambda g: (g // hp, 0, g % hp)),
            const_spec, const_spec, const_spec,
            const_spec, const_spec, const_spec,
        ],
        out_specs=[
            pl.BlockSpec((1, _K, _DH2), lambda g: (g, 0, 0)),
            pl.BlockSpec((1, _K, _DH2), lambda g: (g, 0, 0)),
        ],
        out_shape=[
            jax.ShapeDtypeStruct((_G, _K, _DH2), jnp.float32),
            jax.ShapeDtypeStruct((_G, _K, _DH2), jnp.int32),
        ],
        interpret=interpret,
    )


def _agg_sc_body(vt_hbm, w_hbm, idx_hbm, out_hbm, vpad, wv, iv, ov):
    cid = lax.axis_index("c")
    sid = lax.axis_index("s")
    wid = sid * 2 + cid
    base = wid * _CPW
    lane = lax.iota(jnp.int32, 16)

    def col_body(j, carry):
        col = base + j
        pltpu.sync_copy(vt_hbm.at[col], vpad.at[pl.ds(0, _L)])
        pltpu.sync_copy(w_hbm.at[col], wv)
        pltpu.sync_copy(idx_hbm.at[col], iv)
        tailv = vpad[pl.ds(_L - 16, 16)]
        lastv = jnp.broadcast_to(
            jnp.sum(jnp.where(lane == 15, tailv, 0.0)), (16,))

        def fill(t, c2):
            vpad[pl.ds(_L + t * 16, 16)] = lastv
            return c2

        lax.fori_loop(0, _L // 16, fill, 0)

        wvec = wv[...]
        ivec = iv[...]
        wb = [jnp.broadcast_to(jnp.sum(jnp.where(lane == i, wvec, 0.0)), (16,))
              for i in range(_K)]
        ib = [jnp.broadcast_to(jnp.sum(jnp.where(lane == i, ivec, 0)), (16,))
              for i in range(_K)]

        def t_body(t, c3):
            tvec = lane + t * 16
            acc = wb[0] * plsc.load_gather(vpad, [ib[0] + tvec])
            for i in range(1, _K):
                acc = acc + wb[i] * plsc.load_gather(vpad, [ib[i] + tvec])
            ov[pl.ds(t * 16, 16)] = acc
            return c3

        lax.fori_loop(0, _L // 16, t_body, 0)
        pltpu.sync_copy(ov, out_hbm.at[col])
        return carry

    lax.fori_loop(0, _CPW, col_body, 0)


def _make_agg():
    mesh = plsc.VectorSubcoreMesh(core_axis_name="c", subcore_axis_name="s")
    return pl.kernel(
        _agg_sc_body,
        mesh=mesh,
        compiler_params=pltpu.CompilerParams(needs_layout_passes=False),
        out_type=jax.ShapeDtypeStruct((_NCOLS, _L), jnp.float32),
        scratch_types=[
            pltpu.VMEM((2 * _L,), jnp.float32),
            pltpu.VMEM((_K,), jnp.float32),
            pltpu.VMEM((_K,), jnp.int32),
            pltpu.VMEM((_L,), jnp.float32),
        ],
    )


@jax.jit
def kernel(Q, K, V):
    B, L, DM = Q.shape
    wg, ig = _make_corr_topk()(Q, K, _F1RE, _F1IM, _TRE, _TIM, _TTRE, _TTIM)
    wt = wg.transpose(0, 2, 1).reshape(_NCOLS, _K)
    it = ig.transpose(0, 2, 1).reshape(_NCOLS, _K)
    vt = V.transpose(0, 2, 1).reshape(_NCOLS, L)
    out_cols = _make_agg()(vt, wt, it)
    out = out_cols.reshape(B, DM, L).transpose(0, 2, 1)
    return out


# speedup vs baseline: 10.4846x; 1.1042x over previous
import math

import numpy as np
import jax
import jax.numpy as jnp
from jax import lax
from jax.experimental import pallas as pl
from jax.experimental.pallas import tpu as pltpu
from jax.experimental.pallas import tpu_sc as plsc

_L = 4096
_N = 64
_H = 12
_DM = 768
_DH2 = 128
_G = 4 * _H // 2
_K = 16
_NW = 32
_NCOLS = 4 * _DM
_CPW = _NCOLS // _NW

_tt = np.arange(_N)
_F = np.exp(-2j * np.pi * np.outer(_tt, _tt) / _N)
_T = np.exp(-2j * np.pi * np.outer(_tt, _tt) / _L)
_F1RE = np.ascontiguousarray(_F.real, np.float32)
_F1IM = np.ascontiguousarray(_F.imag, np.float32)
_TRE = np.ascontiguousarray(_T.real, np.float32)
_TIM = np.ascontiguousarray(_T.imag, np.float32)
_TTRE = np.ascontiguousarray(_T.real.T, np.float32)
_TTIM = np.ascontiguousarray(-_T.imag.T, np.float32)


def _fwd_fft(x, fre, fim, tre, tim):
    X = x.reshape(_N, _N * _DH2)
    yre = jnp.dot(fre, X, preferred_element_type=jnp.float32, precision=lax.Precision.HIGHEST)
    yim = jnp.dot(fim, X, preferred_element_type=jnp.float32, precision=lax.Precision.HIGHEST)
    y3re = yre.reshape(_N, _N, _DH2)
    y3im = yim.reshape(_N, _N, _DH2)
    t_re = tre[:, :, None]
    t_im = tim[:, :, None]
    zre = y3re * t_re - y3im * t_im
    zim = y3re * t_im + y3im * t_re
    zre = zre.transpose(1, 0, 2).reshape(_N, _N * _DH2)
    zim = zim.transpose(1, 0, 2).reshape(_N, _N * _DH2)
    wre = (jnp.dot(fre, zre, preferred_element_type=jnp.float32, precision=lax.Precision.HIGHEST)
           - jnp.dot(fim, zim, preferred_element_type=jnp.float32, precision=lax.Precision.HIGHEST))
    wim = (jnp.dot(fre, zim, preferred_element_type=jnp.float32, precision=lax.Precision.HIGHEST)
           + jnp.dot(fim, zre, preferred_element_type=jnp.float32, precision=lax.Precision.HIGHEST))
    return wre, wim


def _inv_fft_real(pre, pim, fre, fim, ttre, ttim):
    gre = (jnp.dot(fre, pre, preferred_element_type=jnp.float32, precision=lax.Precision.HIGHEST)
           + jnp.dot(fim, pim, preferred_element_type=jnp.float32, precision=lax.Precision.HIGHEST))
    gim = (jnp.dot(fre, pim, preferred_element_type=jnp.float32, precision=lax.Precision.HIGHEST)
           - jnp.dot(fim, pre, preferred_element_type=jnp.float32, precision=lax.Precision.HIGHEST))
    g3re = gre.reshape(_N, _N, _DH2)
    g3im = gim.reshape(_N, _N, _DH2)
    t_re = ttre[:, :, None]
    t_im = ttim[:, :, None]
    hre = g3re * t_re - g3im * t_im
    him = g3re * t_im + g3im * t_re
    hre = hre.transpose(1, 0, 2).reshape(_N, _N * _DH2)
    him = him.transpose(1, 0, 2).reshape(_N, _N * _DH2)
    rre = (jnp.dot(fre, hre, preferred_element_type=jnp.float32, precision=lax.Precision.HIGHEST)
           + jnp.dot(fim, him, preferred_element_type=jnp.float32, precision=lax.Precision.HIGHEST))
    return rre.reshape(_L, _DH2) * (1.0 / _L)


def _corr_topk_body(q_ref, k_ref, fre_ref, fim_ref, tre_ref, tim_ref,
                    ttre_ref, ttim_ref, w_ref, i_ref):
    fre = fre_ref[...]
    fim = fim_ref[...]
    tre = tre_ref[...]
    tim = tim_ref[...]
    ttre = ttre_ref[...]
    ttim = ttim_ref[...]

    q = q_ref[0]
    k = k_ref[0]
    qre, qim = _fwd_fft(q, fre, fim, tre, tim)
    kre, kim = _fwd_fft(k, fre, fim, tre, tim)
    pre = qre * kre + qim * kim
    pim = qim * kre - qre * kim
    corr = _inv_fft_real(pre, pim, fre, fim, ttre, ttim)

    iota_t = lax.broadcasted_iota(jnp.int32, (_L, _DH2), 0)
    c = corr
    wrows = []
    irows = []
    for _ in range(_K):
        m = jnp.max(c, axis=0, keepdims=True)
        hit = c == m
        idx = jnp.min(jnp.where(hit, iota_t, _L), axis=0, keepdims=True)
        wrows.append(m)
        irows.append(idx)
        c = jnp.where(iota_t == idx, -jnp.inf, c)
    wmat = jnp.concatenate(wrows, axis=0)
    imat = jnp.concatenate(irows, axis=0)
    wmax = jnp.max(wmat, axis=0, keepdims=True)
    e = jnp.exp(wmat - wmax)
    wsm = e / jnp.sum(e, axis=0, keepdims=True)
    w_ref[0] = wsm
    i_ref[0] = imat


def _make_corr_topk(nb=4, interpret=False):
    hp = _H // 2
    ng = nb * hp
    const_spec = pl.BlockSpec((_N, _N), lambda g: (0, 0))
    return pl.pallas_call(
        _corr_topk_body,
        grid=(ng,),
        in_specs=[
            pl.BlockSpec((1, _L, _DH2), lambda g: (g // hp, 0, g % hp)),
            pl.BlockSpec((1, _L, _DH2), lambda g: (g // hp, 0, g % hp)),
            const_spec, const_spec, const_spec,
            const_spec, const_spec, const_spec,
        ],
        out_specs=[
            pl.BlockSpec((1, _K, _DH2), lambda g: (g, 0, 0)),
            pl.BlockSpec((1, _K, _DH2), lambda g: (g, 0, 0)),
        ],
        out_shape=[
            jax.ShapeDtypeStruct((ng, _K, _DH2), jnp.float32),
            jax.ShapeDtypeStruct((ng, _K, _DH2), jnp.int32),
        ],
        interpret=interpret,
    )


def _agg_sc_body(cpw, vt_hbm, w_hbm, idx_hbm, out_hbm, vpad, wv, iv, ov):
    cid = lax.axis_index("c")
    sid = lax.axis_index("s")
    wid = sid * 2 + cid
    base = wid * cpw
    lane = lax.iota(jnp.int32, 16)

    def col_body(j, carry):
        col = base + j
        pltpu.sync_copy(vt_hbm.at[col], vpad.at[pl.ds(0, _L)])
        pltpu.sync_copy(w_hbm.at[col], wv)
        pltpu.sync_copy(idx_hbm.at[col], iv)
        tailv = vpad[pl.ds(_L - 16, 16)]
        lastv = jnp.broadcast_to(
            jnp.sum(jnp.where(lane == 15, tailv, 0.0)), (16,))

        def fill(t, c2):
            vpad[pl.ds(_L + t * 16, 16)] = lastv
            return c2

        lax.fori_loop(0, _L // 16, fill, 0)

        wvec = wv[...]
        ivec = iv[...]
        wb = [jnp.broadcast_to(jnp.sum(jnp.where(lane == i, wvec, 0.0)), (16,))
              for i in range(_K)]
        ib = [jnp.broadcast_to(jnp.sum(jnp.where(lane == i, ivec, 0)), (16,))
              for i in range(_K)]

        def t_body(t, c3):
            tvec = lane + t * 16
            acc = wb[0] * plsc.load_gather(vpad, [ib[0] + tvec])
            for i in range(1, _K):
                acc = acc + wb[i] * plsc.load_gather(vpad, [ib[i] + tvec])
            ov[pl.ds(t * 16, 16)] = acc
            return c3

        lax.fori_loop(0, _L // 16, t_body, 0)
        pltpu.sync_copy(ov, out_hbm.at[col])
        return carry

    lax.fori_loop(0, cpw, col_body, 0)


def _make_agg(ncols):
    import functools
    mesh = plsc.VectorSubcoreMesh(core_axis_name="c", subcore_axis_name="s")
    return pl.kernel(
        functools.partial(_agg_sc_body, ncols // _NW),
        mesh=mesh,
        compiler_params=pltpu.CompilerParams(needs_layout_passes=False),
        out_type=jax.ShapeDtypeStruct((ncols, _L), jnp.float32),
        scratch_types=[
            pltpu.VMEM((2 * _L,), jnp.float32),
            pltpu.VMEM((_K,), jnp.float32),
            pltpu.VMEM((_K,), jnp.int32),
            pltpu.VMEM((_L,), jnp.float32),
        ],
    )


@jax.jit
def kernel(Q, K, V):
    B, L, DM = Q.shape
    nb = 2
    outs = []
    for g0 in range(0, B, nb):
        q = lax.slice_in_dim(Q, g0, g0 + nb, axis=0)
        k = lax.slice_in_dim(K, g0, g0 + nb, axis=0)
        v = lax.slice_in_dim(V, g0, g0 + nb, axis=0)
        wg, ig = _make_corr_topk(nb)(q, k, _F1RE, _F1IM, _TRE, _TIM,
                                     _TTRE, _TTIM)
        ncols = nb * DM
        wt = wg.transpose(0, 2, 1).reshape(ncols, _K)
        it = ig.transpose(0, 2, 1).reshape(ncols, _K)
        vt = v.transpose(0, 2, 1).reshape(ncols, L)
        outs.append(_make_agg(ncols)(vt, wt, it))
    out_cols = jnp.concatenate(outs, axis=0)
    out = out_cols.reshape(B, DM, L).transpose(0, 2, 1)
    return out


# speedup vs baseline: 11.6149x; 1.1078x over previous
import math

import numpy as np
import jax
import jax.numpy as jnp
from jax import lax
from jax.experimental import pallas as pl
from jax.experimental.pallas import tpu as pltpu
from jax.experimental.pallas import tpu_sc as plsc

_L = 4096
_N = 64
_H = 12
_DM = 768
_DH2 = 128
_G = 4 * _H // 2
_K = 16
_NW = 32
_NCOLS = 4 * _DM
_CPW = _NCOLS // _NW

_tt = np.arange(_N)
_F = np.exp(-2j * np.pi * np.outer(_tt, _tt) / _N)
_T = np.exp(-2j * np.pi * np.outer(_tt, _tt) / _L)
_F1RE = np.ascontiguousarray(_F.real, np.float32)
_F1IM = np.ascontiguousarray(_F.imag, np.float32)
_TRE = np.ascontiguousarray(_T.real, np.float32)
_TIM = np.ascontiguousarray(_T.imag, np.float32)
_TTRE = np.ascontiguousarray(_T.real.T, np.float32)
_TTIM = np.ascontiguousarray(-_T.imag.T, np.float32)


def _fwd_fft(x, fre, fim, tre, tim):
    X = x.reshape(_N, _N * _DH2)
    yre = jnp.dot(fre, X, preferred_element_type=jnp.float32, precision=lax.Precision.HIGHEST)
    yim = jnp.dot(fim, X, preferred_element_type=jnp.float32, precision=lax.Precision.HIGHEST)
    y3re = yre.reshape(_N, _N, _DH2)
    y3im = yim.reshape(_N, _N, _DH2)
    t_re = tre[:, :, None]
    t_im = tim[:, :, None]
    zre = y3re * t_re - y3im * t_im
    zim = y3re * t_im + y3im * t_re
    zre = zre.transpose(1, 0, 2).reshape(_N, _N * _DH2)
    zim = zim.transpose(1, 0, 2).reshape(_N, _N * _DH2)
    wre = (jnp.dot(fre, zre, preferred_element_type=jnp.float32, precision=lax.Precision.HIGHEST)
           - jnp.dot(fim, zim, preferred_element_type=jnp.float32, precision=lax.Precision.HIGHEST))
    wim = (jnp.dot(fre, zim, preferred_element_type=jnp.float32, precision=lax.Precision.HIGHEST)
           + jnp.dot(fim, zre, preferred_element_type=jnp.float32, precision=lax.Precision.HIGHEST))
    return wre, wim


def _inv_fft_real(pre, pim, fre, fim, ttre, ttim):
    gre = (jnp.dot(fre, pre, preferred_element_type=jnp.float32, precision=lax.Precision.HIGHEST)
           + jnp.dot(fim, pim, preferred_element_type=jnp.float32, precision=lax.Precision.HIGHEST))
    gim = (jnp.dot(fre, pim, preferred_element_type=jnp.float32, precision=lax.Precision.HIGHEST)
           - jnp.dot(fim, pre, preferred_element_type=jnp.float32, precision=lax.Precision.HIGHEST))
    g3re = gre.reshape(_N, _N, _DH2)
    g3im = gim.reshape(_N, _N, _DH2)
    t_re = ttre[:, :, None]
    t_im = ttim[:, :, None]
    hre = g3re * t_re - g3im * t_im
    him = g3re * t_im + g3im * t_re
    hre = hre.transpose(1, 0, 2).reshape(_N, _N * _DH2)
    him = him.transpose(1, 0, 2).reshape(_N, _N * _DH2)
    rre = (jnp.dot(fre, hre, preferred_element_type=jnp.float32, precision=lax.Precision.HIGHEST)
           + jnp.dot(fim, him, preferred_element_type=jnp.float32, precision=lax.Precision.HIGHEST))
    return rre.reshape(_L, _DH2) * (1.0 / _L)


def _corr_topk_body(q_ref, k_ref, fre_ref, fim_ref, tre_ref, tim_ref,
                    ttre_ref, ttim_ref, w_ref, i_ref):
    fre = fre_ref[...]
    fim = fim_ref[...]
    tre = tre_ref[...]
    tim = tim_ref[...]
    ttre = ttre_ref[...]
    ttim = ttim_ref[...]

    q = q_ref[0]
    k = k_ref[0]
    qre, qim = _fwd_fft(q, fre, fim, tre, tim)
    kre, kim = _fwd_fft(k, fre, fim, tre, tim)
    pre = qre * kre + qim * kim
    pim = qim * kre - qre * kim
    corr = _inv_fft_real(pre, pim, fre, fim, ttre, ttim)

    iota_t = lax.broadcasted_iota(jnp.int32, (_L, _DH2), 0)
    c = corr
    wrows = []
    irows = []
    for _ in range(_K):
        m = jnp.max(c, axis=0, keepdims=True)
        hit = c == m
        idx = jnp.min(jnp.where(hit, iota_t, _L), axis=0, keepdims=True)
        wrows.append(m)
        irows.append(idx)
        c = jnp.where(iota_t == idx, -jnp.inf, c)
    wmat = jnp.concatenate(wrows, axis=0)
    imat = jnp.concatenate(irows, axis=0)
    wmax = jnp.max(wmat, axis=0, keepdims=True)
    e = jnp.exp(wmat - wmax)
    wsm = e / jnp.sum(e, axis=0, keepdims=True)
    w_ref[0] = wsm
    i_ref[0] = imat


def _make_corr_topk(nb=4, interpret=False):
    hp = _H // 2
    ng = nb * hp
    const_spec = pl.BlockSpec((_N, _N), lambda g: (0, 0))
    return pl.pallas_call(
        _corr_topk_body,
        grid=(ng,),
        in_specs=[
            pl.BlockSpec((1, _L, _DH2), lambda g: (g // hp, 0, g % hp)),
            pl.BlockSpec((1, _L, _DH2), lambda g: (g // hp, 0, g % hp)),
            const_spec, const_spec, const_spec,
            const_spec, const_spec, const_spec,
        ],
        out_specs=[
            pl.BlockSpec((1, _K, _DH2), lambda g: (g, 0, 0)),
            pl.BlockSpec((1, _K, _DH2), lambda g: (g, 0, 0)),
        ],
        out_shape=[
            jax.ShapeDtypeStruct((ng, _K, _DH2), jnp.float32),
            jax.ShapeDtypeStruct((ng, _K, _DH2), jnp.int32),
        ],
        interpret=interpret,
    )


def _agg_sc_body(cpw, vt_hbm, w_hbm, idx_hbm, out_hbm, vpad, wv, iv, ov):
    cid = lax.axis_index("c")
    sid = lax.axis_index("s")
    wid = sid * 2 + cid
    base = wid * cpw
    lane = lax.iota(jnp.int32, 16)

    def col_body(j, carry):
        col = base + j
        pltpu.sync_copy(vt_hbm.at[col], vpad.at[pl.ds(0, _L)])
        pltpu.sync_copy(w_hbm.at[col], wv)
        pltpu.sync_copy(idx_hbm.at[col], iv)
        tailv = vpad[pl.ds(_L - 16, 16)]
        lastv = jnp.broadcast_to(
            jnp.sum(jnp.where(lane == 15, tailv, 0.0)), (16,))

        @plsc.parallel_loop(0, _L // 16, unroll=4)
        def fill(t):
            vpad[pl.ds(_L + t * 16, 16)] = lastv

        wvec = wv[...]
        ivec = iv[...]
        wb = [jnp.broadcast_to(jnp.sum(jnp.where(lane == i, wvec, 0.0)), (16,))
              for i in range(_K)]
        ib = [jnp.broadcast_to(jnp.sum(jnp.where(lane == i, ivec, 0)), (16,))
              for i in range(_K)]

        @plsc.parallel_loop(0, _L // 16, unroll=2)
        def t_body(t):
            tvec = lane + t * 16
            acc = wb[0] * plsc.load_gather(vpad, [ib[0] + tvec])
            for i in range(1, _K):
                acc = acc + wb[i] * plsc.load_gather(vpad, [ib[i] + tvec])
            ov[pl.ds(t * 16, 16)] = acc
        pltpu.sync_copy(ov, out_hbm.at[col])
        return carry

    lax.fori_loop(0, cpw, col_body, 0)


def _make_agg(ncols):
    import functools
    mesh = plsc.VectorSubcoreMesh(core_axis_name="c", subcore_axis_name="s")
    return pl.kernel(
        functools.partial(_agg_sc_body, ncols // _NW),
        mesh=mesh,
        compiler_params=pltpu.CompilerParams(needs_layout_passes=False),
        out_type=jax.ShapeDtypeStruct((ncols, _L), jnp.float32),
        scratch_types=[
            pltpu.VMEM((2 * _L,), jnp.float32),
            pltpu.VMEM((_K,), jnp.float32),
            pltpu.VMEM((_K,), jnp.int32),
            pltpu.VMEM((_L,), jnp.float32),
        ],
    )


@jax.jit
def kernel(Q, K, V):
    B, L, DM = Q.shape
    nb = 2
    outs = []
    for g0 in range(0, B, nb):
        q = lax.slice_in_dim(Q, g0, g0 + nb, axis=0)
        k = lax.slice_in_dim(K, g0, g0 + nb, axis=0)
        v = lax.slice_in_dim(V, g0, g0 + nb, axis=0)
        wg, ig = _make_corr_topk(nb)(q, k, _F1RE, _F1IM, _TRE, _TIM,
                                     _TTRE, _TTIM)
        ncols = nb * DM
        wt = wg.transpose(0, 2, 1).reshape(ncols, _K)
        it = ig.transpose(0, 2, 1).reshape(ncols, _K)
        vt = v.transpose(0, 2, 1).reshape(ncols, L)
        outs.append(_make_agg(ncols)(vt, wt, it))
    out_cols = jnp.concatenate(outs, axis=0)
    out = out_cols.reshape(B, DM, L).transpose(0, 2, 1)
    return out
